# baseline (speedup 1.0000x reference)
import functools

import jax
import jax.numpy as jnp
from jax import lax
from jax.experimental import pallas as pl
from jax.experimental.pallas import tpu as pltpu
from jax.experimental.pallas import tpu_sc as plsc

_NC, _NS = 2, 16
_NW = _NC * _NS
_BW = 128
_L = 16
_NG = _BW // _L


def _make(batch, seq, d):
    th_n, hi_n = d // 8, 8
    tb_n = batch // _BW

    mesh = plsc.VectorSubcoreMesh(
        core_axis_name="c", subcore_axis_name="s",
        num_cores=_NC, num_subcores=_NS)

    @functools.partial(
        pl.kernel,
        mesh=mesh,
        compiler_params=pltpu.CompilerParams(
            use_tc_tiling_on_sc=False, needs_layout_passes=False),
        out_type=jax.ShapeDtypeStruct((seq, th_n, tb_n, hi_n, _BW),
                                      jnp.float32),
        scratch_types=[
            pltpu.VMEM((_BW,), jnp.int32),
            pltpu.VMEM((_BW,), jnp.int32),
            pltpu.VMEM((2, _BW, d), jnp.float32),
            pltpu.VMEM((2, d, _BW), jnp.float32),
            pltpu.SemaphoreType.DMA,
            pltpu.SemaphoreType.DMA,
            pltpu.SemaphoreType.DMA,
            pltpu.SemaphoreType.DMA,
            pltpu.SemaphoreType.DMA,
            pltpu.SemaphoreType.DMA,
        ],
    )
    def gather_kernel(idx_hbm, table_hbm, out_hbm,
                      pidx0, pidx1, raw_v, slab_v,
                      psem0, psem1, gsem0, gsem1, osem0, osem1):
        wid = lax.axis_index("s") * _NC + lax.axis_index("c")
        wb = wid * _BW
        pidx = (pidx0, pidx1)
        psem = (psem0, psem1)
        gsem = (gsem0, gsem1)
        osem = (osem0, osem1)

        def fire_pidx(s, a):
            pltpu.async_copy(
                idx_hbm.at[pl.ds(s * batch + wb, _BW)], pidx[a], psem[a])

        def wait_pidx(a):
            pltpu.make_async_copy(
                idx_hbm.at[pl.ds(0, _BW)], pidx[a], psem[a]).wait()

        def fire_gather(a):
            pltpu.async_copy(table_hbm.at[pidx[a]], raw_v.at[a], gsem[a])

        def wait_gather(a):
            pltpu.make_async_copy(
                table_hbm.at[pl.ds(0, _BW)], raw_v.at[a], gsem[a]).wait()

        def fire_out(s, a):
            for th in range(th_n):
                pltpu.async_copy(
                    slab_v.at[a].at[pl.ds(th * hi_n, hi_n)],
                    out_hbm.at[s, th, wid], osem[a])

        def wait_out(a):
            for th in range(th_n):
                pltpu.make_async_copy(
                    slab_v.at[a].at[pl.ds(th * hi_n, hi_n)],
                    out_hbm.at[0, th, 0], osem[a]).wait()

        lanes = lax.iota(jnp.int32, _L)
        bidx = [lanes + bg * _L for bg in range(_NG)]

        def transpose(a):
            def h0body(i, col):
                col2 = lax.bitwise_and(col + 1, d - 1)
                vals = [plsc.load_gather(raw_v.at[a], [bidx[bg], col])
                        for bg in range(_NG)]
                vals2 = [plsc.load_gather(raw_v.at[a], [bidx[bg], col2])
                         for bg in range(_NG)]
                for bg in range(_NG):
                    plsc.store_scatter(
                        slab_v.at[a], [col, bidx[bg]], vals[bg])
                for bg in range(_NG):
                    plsc.store_scatter(
                        slab_v.at[a], [col2, bidx[bg]], vals2[bg])
                return lax.bitwise_and(col2 + 1, d - 1)

            lax.fori_loop(0, d // 2, h0body, lanes)

        def step(s, a, fire_g=True, fire_p=True, wait_o=True):
            b = 1 - a
            if fire_g:
                wait_pidx(b)
                fire_gather(b)
            wait_gather(a)
            if fire_p:
                fire_pidx(s + 2, a)
            if wait_o:
                wait_out(a)
            transpose(a)
            fire_out(s, a)

        fire_pidx(0, 0)
        fire_pidx(1, 1)
        wait_pidx(0)
        fire_gather(0)
        step(0, 0, wait_o=False)
        step(1, 1, wait_o=False)

        def body(i, carry):
            step(2 * i + 2, 0)
            step(2 * i + 3, 1)
            return carry

        lax.fori_loop(0, (seq - 4) // 2, body, 0)

        step(seq - 2, 0, fire_p=False)
        step(seq - 1, 1, fire_g=False, fire_p=False)
        wait_out(0)
        wait_out(1)

    return gather_kernel


def _make_table(v, d):
    nb = v // 128
    per_w = nb // _NW
    extra = nb - per_w * _NW

    mesh = plsc.VectorSubcoreMesh(
        core_axis_name="c", subcore_axis_name="s",
        num_cores=_NC, num_subcores=_NS)

    @functools.partial(
        pl.kernel,
        mesh=mesh,
        compiler_params=pltpu.CompilerParams(
            use_tc_tiling_on_sc=True, needs_layout_passes=False),
        out_type=jax.ShapeDtypeStruct((v * d,), jnp.float32),
        scratch_types=[
            pltpu.VMEM((2, 64, 128), jnp.float32),
            pltpu.VMEM((64 * 128,), jnp.float32),
            pltpu.VMEM((64 * 128,), jnp.float32),
            pltpu.VMEM((4096,), jnp.float32),
            pltpu.SemaphoreType.DMA,
            pltpu.SemaphoreType.DMA,
            pltpu.SemaphoreType.DMA,
            pltpu.SemaphoreType.DMA,
            pltpu.SemaphoreType.DMA,
        ],
    )
    def tr_kernel(wt_hbm, tail_hbm, out_hbm, vin, vout0, vout1, tl_v,
                  isem0, isem1, osem0, osem1, tsem):
        wid = lax.axis_index("s") * _NC + lax.axis_index("c")
        base = wid * per_w + jnp.minimum(wid, extra)
        isem = (isem0, isem1)
        osem = (osem0, osem1)
        vout = (vout0, vout1)

        def fire_in(blk, a):
            pltpu.async_copy(
                wt_hbm.at[:, pl.ds((base + blk) * 128, 128)], vin.at[a],
                isem[a])

        def wait_in(a):
            pltpu.make_async_copy(
                wt_hbm.at[:, pl.ds(0, 128)], vin.at[a], isem[a]).wait()

        def fire_out(blk, a):
            pltpu.async_copy(
                vout[a],
                out_hbm.at[pl.ds((base + blk) * (64 * 128), 64 * 128)],
                osem[a])

        def wait_out(a):
            pltpu.make_async_copy(
                vout[a], out_hbm.at[pl.ds(0, 64 * 128)], osem[a]).wait()

        lanes = lax.iota(jnp.int32, _L)
        jidx = [lanes + jg * _L for jg in range(8)]
        j64 = [lax.shift_left(j, 6) for j in jidx]

        def transpose(a):
            def h0body(i, col):
                col2 = lax.bitwise_and(col + 1, d - 1)
                vals = [plsc.load_gather(vin.at[a], [col, jidx[jg]])
                        for jg in range(8)]
                vals2 = [plsc.load_gather(vin.at[a], [col2, jidx[jg]])
                         for jg in range(8)]
                for jg in range(8):
                    plsc.store_scatter(vout[a], [j64[jg] + col], vals[jg])
                for jg in range(8):
                    plsc.store_scatter(vout[a], [j64[jg] + col2], vals2[jg])
                return lax.bitwise_and(col2 + 1, d - 1)

            lax.fori_loop(0, d // 2, h0body, lanes)

        def step(blk, a, fire_nxt=True, wait_o=True):
            b = 1 - a
            if fire_nxt:
                pl.when(blk + 1 < per_w + (wid < extra))(
                    lambda: fire_in(blk + 1, b))
            wait_in(a)
            if wait_o:
                wait_out(a)
            transpose(a)
            fire_out(blk, a)

        @pl.when(wid == 0)
        def _():
            pltpu.async_copy(tail_hbm, tl_v, tsem)
            pltpu.make_async_copy(tail_hbm, tl_v, tsem).wait()
            pltpu.async_copy(
                tl_v, out_hbm.at[pl.ds((v // 128) * 128 * d, (v % 128) * d)],
                tsem)
            pltpu.make_async_copy(
                tl_v, out_hbm.at[pl.ds(0, (v % 128) * d)], tsem).wait()

        fire_in(0, 0)
        step(0, 0, wait_o=False)
        step(1, 1, wait_o=False)

        def body(i, carry):
            step(2 * i + 2, 0)
            step(2 * i + 3, 1)
            return carry

        lax.fori_loop(0, (per_w - 4) // 2, body, 0)

        step(per_w - 2, 0)
        step(per_w - 1, 1)

        @pl.when(wid < extra)
        def _():
            wait_in(0)
            wait_out(0)
            transpose(0)
            fire_out(per_w, 0)
            wait_out(0)
        pl.when(wid >= extra)(lambda: wait_out(0))
        wait_out(1)

    return tr_kernel


def kernel(input_ids, word_embeddings):
    batch, seq = input_ids.shape
    v, d = word_embeddings.shape
    idx_flat = input_ids.T.reshape(-1).astype(jnp.int32)
    wt = word_embeddings.T
    tail2 = word_embeddings[(v // 128) * 128:].reshape(-1)
    table2 = _make_table(v, d)(wt, tail2).reshape(v, d)
    out5 = _make(batch, seq, d)(idx_flat, table2)
    out = out5.transpose(2, 4, 0, 1, 3).reshape(batch, seq, d)
    return out


# speedup vs baseline: 1.1010x; 1.1010x over previous
import functools

import jax
import jax.numpy as jnp
from jax import lax
from jax.experimental import pallas as pl
from jax.experimental.pallas import tpu as pltpu
from jax.experimental.pallas import tpu_sc as plsc

_NC, _NS = 2, 16
_NW = _NC * _NS
_BW = 128
_L = 16
_NG = _BW // _L


def _make(batch, seq, d):
    th_n, hi_n = d // 8, 8
    tb_n = batch // _BW

    mesh = plsc.VectorSubcoreMesh(
        core_axis_name="c", subcore_axis_name="s",
        num_cores=_NC, num_subcores=_NS)

    @functools.partial(
        pl.kernel,
        mesh=mesh,
        compiler_params=pltpu.CompilerParams(
            use_tc_tiling_on_sc=False, needs_layout_passes=False),
        out_type=jax.ShapeDtypeStruct((seq, th_n, tb_n, hi_n, _BW),
                                      jnp.float32),
        scratch_types=[
            pltpu.VMEM((_BW,), jnp.int32),
            pltpu.VMEM((_BW,), jnp.int32),
            pltpu.VMEM((2, _BW, d), jnp.float32),
            pltpu.VMEM((2, d, _BW), jnp.float32),
            pltpu.SemaphoreType.DMA,
            pltpu.SemaphoreType.DMA,
            pltpu.SemaphoreType.DMA,
            pltpu.SemaphoreType.DMA,
            pltpu.SemaphoreType.DMA,
            pltpu.SemaphoreType.DMA,
        ],
    )
    def gather_kernel(idx_hbm, table_hbm, out_hbm,
                      pidx0, pidx1, raw_v, slab_v,
                      psem0, psem1, gsem0, gsem1, osem0, osem1):
        wid = lax.axis_index("s") * _NC + lax.axis_index("c")
        wb = wid * _BW
        pidx = (pidx0, pidx1)
        psem = (psem0, psem1)
        gsem = (gsem0, gsem1)
        osem = (osem0, osem1)

        def fire_pidx(s, a):
            pltpu.async_copy(
                idx_hbm.at[pl.ds(s * batch + wb, _BW)], pidx[a], psem[a])

        def wait_pidx(a):
            pltpu.make_async_copy(
                idx_hbm.at[pl.ds(0, _BW)], pidx[a], psem[a]).wait()

        def fire_gather(a):
            pltpu.async_copy(table_hbm.at[pidx[a]], raw_v.at[a], gsem[a])

        def wait_gather(a):
            pltpu.make_async_copy(
                table_hbm.at[pl.ds(0, _BW)], raw_v.at[a], gsem[a]).wait()

        def fire_out(s, a):
            for th in range(th_n):
                pltpu.async_copy(
                    slab_v.at[a].at[pl.ds(th * hi_n, hi_n)],
                    out_hbm.at[s, th, wid], osem[a])

        def wait_out(a):
            for th in range(th_n):
                pltpu.make_async_copy(
                    slab_v.at[a].at[pl.ds(th * hi_n, hi_n)],
                    out_hbm.at[0, th, 0], osem[a]).wait()

        lanes = lax.iota(jnp.int32, _L)
        bidx = [lanes + bg * _L for bg in range(_NG)]

        def transpose(a):
            def h0body(i, col):
                col2 = lax.bitwise_and(col + 1, d - 1)
                vals = [plsc.load_gather(raw_v.at[a], [bidx[bg], col])
                        for bg in range(_NG)]
                vals2 = [plsc.load_gather(raw_v.at[a], [bidx[bg], col2])
                         for bg in range(_NG)]
                for bg in range(_NG):
                    plsc.store_scatter(
                        slab_v.at[a], [col, bidx[bg]], vals[bg])
                for bg in range(_NG):
                    plsc.store_scatter(
                        slab_v.at[a], [col2, bidx[bg]], vals2[bg])
                return lax.bitwise_and(col2 + 1, d - 1)

            lax.fori_loop(0, d // 2, h0body, lanes)

        def step(s, a, fire_g=True, fire_p=True, wait_o=True):
            b = 1 - a
            if fire_g:
                wait_pidx(b)
                fire_gather(b)
            wait_gather(a)
            if fire_p:
                fire_pidx(s + 2, a)
            if wait_o:
                wait_out(a)
            transpose(a)
            fire_out(s, a)

        fire_pidx(0, 0)
        fire_pidx(1, 1)
        wait_pidx(0)
        fire_gather(0)
        step(0, 0, wait_o=False)
        step(1, 1, wait_o=False)

        def body(i, carry):
            step(2 * i + 2, 0)
            step(2 * i + 3, 1)
            return carry

        lax.fori_loop(0, (seq - 4) // 2, body, 0)

        step(seq - 2, 0, fire_p=False)
        step(seq - 1, 1, fire_g=False, fire_p=False)
        wait_out(0)
        wait_out(1)

    return gather_kernel


def _make_table(v, d):
    nb = v // 256
    per_w = nb // _NW
    extra = nb - per_w * _NW

    mesh = plsc.VectorSubcoreMesh(
        core_axis_name="c", subcore_axis_name="s",
        num_cores=_NC, num_subcores=_NS)

    @functools.partial(
        pl.kernel,
        mesh=mesh,
        compiler_params=pltpu.CompilerParams(
            use_tc_tiling_on_sc=True, needs_layout_passes=False),
        out_type=jax.ShapeDtypeStruct((v * d,), jnp.float32),
        scratch_types=[
            pltpu.VMEM((2, 64, 256), jnp.float32),
            pltpu.VMEM((64 * 256,), jnp.float32),
            pltpu.VMEM((64 * 256,), jnp.float32),
            pltpu.VMEM((4096,), jnp.float32),
            pltpu.SemaphoreType.DMA,
            pltpu.SemaphoreType.DMA,
            pltpu.SemaphoreType.DMA,
            pltpu.SemaphoreType.DMA,
            pltpu.SemaphoreType.DMA,
        ],
    )
    def tr_kernel(wt_hbm, tail_hbm, out_hbm, vin, vout0, vout1, tl_v,
                  isem0, isem1, osem0, osem1, tsem):
        wid = lax.axis_index("s") * _NC + lax.axis_index("c")
        base = wid * per_w + jnp.minimum(wid, extra)
        isem = (isem0, isem1)
        osem = (osem0, osem1)
        vout = (vout0, vout1)

        def fire_in(blk, a):
            pltpu.async_copy(
                wt_hbm.at[:, pl.ds((base + blk) * 256, 256)], vin.at[a],
                isem[a])

        def wait_in(a):
            pltpu.make_async_copy(
                wt_hbm.at[:, pl.ds(0, 256)], vin.at[a], isem[a]).wait()

        def fire_out(blk, a):
            pltpu.async_copy(
                vout[a],
                out_hbm.at[pl.ds((base + blk) * (64 * 256), 64 * 256)],
                osem[a])

        def wait_out(a):
            pltpu.make_async_copy(
                vout[a], out_hbm.at[pl.ds(0, 64 * 256)], osem[a]).wait()

        lanes = lax.iota(jnp.int32, _L)
        jidx = [lanes + jg * _L for jg in range(16)]
        j64 = [lax.shift_left(j, 6) for j in jidx]

        def transpose(a):
            def h0body(i, col):
                col2 = lax.bitwise_and(col + 1, d - 1)
                for cl, half in ((col, 0), (col, 1), (col2, 0), (col2, 1)):
                    vals = [plsc.load_gather(
                        vin.at[a], [cl, jidx[half * 8 + jg]])
                        for jg in range(8)]
                    for jg in range(8):
                        plsc.store_scatter(
                            vout[a], [j64[half * 8 + jg] + cl], vals[jg])
                return lax.bitwise_and(col2 + 1, d - 1)

            lax.fori_loop(0, d // 2, h0body, lanes)

        def step(blk, a, fire_nxt=True, wait_o=True):
            b = 1 - a
            if fire_nxt:
                pl.when(blk + 1 < per_w + (wid < extra))(
                    lambda: fire_in(blk + 1, b))
            wait_in(a)
            if wait_o:
                wait_out(a)
            transpose(a)
            fire_out(blk, a)

        @pl.when(wid == 0)
        def _():
            pltpu.async_copy(tail_hbm, tl_v, tsem)
            pltpu.make_async_copy(tail_hbm, tl_v, tsem).wait()
            pltpu.async_copy(
                tl_v, out_hbm.at[pl.ds((v // 256) * 256 * d, (v % 256) * d)],
                tsem)
            pltpu.make_async_copy(
                tl_v, out_hbm.at[pl.ds(0, (v % 256) * d)], tsem).wait()

        fire_in(0, 0)
        step(0, 0, wait_o=False)
        step(1, 1, wait_o=False)

        def body(i, carry):
            step(2 * i + 2, 0)
            step(2 * i + 3, 1)
            return carry

        lax.fori_loop(0, (per_w - 4) // 2, body, 0)

        step(per_w - 2, 0)
        step(per_w - 1, 1)

        @pl.when(wid < extra)
        def _():
            wait_in(0)
            wait_out(0)
            transpose(0)
            fire_out(per_w, 0)
            wait_out(0)
        pl.when(wid >= extra)(lambda: wait_out(0))
        wait_out(1)

    return tr_kernel


def kernel(input_ids, word_embeddings):
    batch, seq = input_ids.shape
    v, d = word_embeddings.shape
    idx_flat = input_ids.T.reshape(-1).astype(jnp.int32)
    wt = word_embeddings.T
    tail2 = word_embeddings[(v // 128) * 128:].reshape(-1)
    table2 = _make_table(v, d)(wt, tail2).reshape(v, d)
    out5 = _make(batch, seq, d)(idx_flat, table2)
    out = out5.transpose(2, 4, 0, 1, 3).reshape(batch, seq, d)
    return out


# speedup vs baseline: 1.1070x; 1.0055x over previous
import functools

import jax
import jax.numpy as jnp
from jax import lax
from jax.experimental import pallas as pl
from jax.experimental.pallas import tpu as pltpu
from jax.experimental.pallas import tpu_sc as plsc

_NC, _NS = 2, 16
_NW = _NC * _NS
_BW = 128
_L = 16
_NG = _BW // _L


def _make(batch, seq, d):
    th_n, hi_n = d // 8, 8
    tb_n = batch // _BW

    mesh = plsc.VectorSubcoreMesh(
        core_axis_name="c", subcore_axis_name="s",
        num_cores=_NC, num_subcores=_NS)

    @functools.partial(
        pl.kernel,
        mesh=mesh,
        compiler_params=pltpu.CompilerParams(
            use_tc_tiling_on_sc=False, needs_layout_passes=False),
        out_type=jax.ShapeDtypeStruct((seq, th_n, tb_n, hi_n, _BW),
                                      jnp.float32),
        scratch_types=[
            pltpu.VMEM((_BW,), jnp.int32),
            pltpu.VMEM((_BW,), jnp.int32),
            pltpu.VMEM((2, _BW, d), jnp.float32),
            pltpu.VMEM((2, th_n, hi_n, _BW), jnp.float32),
            pltpu.SemaphoreType.DMA,
            pltpu.SemaphoreType.DMA,
            pltpu.SemaphoreType.DMA,
            pltpu.SemaphoreType.DMA,
            pltpu.SemaphoreType.DMA,
            pltpu.SemaphoreType.DMA,
        ],
    )
    def gather_kernel(idx_hbm, table_hbm, out_hbm,
                      pidx0, pidx1, raw_v, slab_v,
                      psem0, psem1, gsem0, gsem1, osem0, osem1):
        wid = lax.axis_index("s") * _NC + lax.axis_index("c")
        wb = wid * _BW
        pidx = (pidx0, pidx1)
        psem = (psem0, psem1)
        gsem = (gsem0, gsem1)
        osem = (osem0, osem1)

        def fire_pidx(s, a):
            pltpu.async_copy(
                idx_hbm.at[pl.ds(s * batch + wb, _BW)], pidx[a], psem[a])

        def wait_pidx(a):
            pltpu.make_async_copy(
                idx_hbm.at[pl.ds(0, _BW)], pidx[a], psem[a]).wait()

        def fire_gather(a):
            pltpu.async_copy(table_hbm.at[pidx[a]], raw_v.at[a], gsem[a])

        def wait_gather(a):
            pltpu.make_async_copy(
                table_hbm.at[pl.ds(0, _BW)], raw_v.at[a], gsem[a]).wait()

        def fire_out(s, a):
            pltpu.async_copy(
                slab_v.at[a], out_hbm.at[s].at[:, wid], osem[a])

        def wait_out(a):
            pltpu.make_async_copy(
                slab_v.at[a], out_hbm.at[0].at[:, 0], osem[a]).wait()

        lanes = lax.iota(jnp.int32, _L)
        bidx = [lanes + bg * _L for bg in range(_NG)]

        def transpose(a):
            def h0body(i, col):
                col2 = lax.bitwise_and(col + 1, d - 1)
                vals = [plsc.load_gather(raw_v.at[a], [bidx[bg], col])
                        for bg in range(_NG)]
                vals2 = [plsc.load_gather(raw_v.at[a], [bidx[bg], col2])
                         for bg in range(_NG)]
                th1 = lax.shift_right_logical(col, 3)
                hi1 = lax.bitwise_and(col, 7)
                th2 = lax.shift_right_logical(col2, 3)
                hi2 = lax.bitwise_and(col2, 7)
                for bg in range(_NG):
                    plsc.store_scatter(
                        slab_v.at[a], [th1, hi1, bidx[bg]], vals[bg])
                for bg in range(_NG):
                    plsc.store_scatter(
                        slab_v.at[a], [th2, hi2, bidx[bg]], vals2[bg])
                return lax.bitwise_and(col2 + 1, d - 1)

            lax.fori_loop(0, d // 2, h0body, lanes)

        def step(s, a, fire_g=True, fire_p=True, wait_o=True):
            b = 1 - a
            if fire_g:
                wait_pidx(b)
                fire_gather(b)
            wait_gather(a)
            if fire_p:
                fire_pidx(s + 2, a)
            if wait_o:
                wait_out(a)
            transpose(a)
            fire_out(s, a)

        fire_pidx(0, 0)
        fire_pidx(1, 1)
        wait_pidx(0)
        fire_gather(0)
        step(0, 0, wait_o=False)
        step(1, 1, wait_o=False)

        def body(i, carry):
            step(2 * i + 2, 0)
            step(2 * i + 3, 1)
            return carry

        lax.fori_loop(0, (seq - 4) // 2, body, 0)

        step(seq - 2, 0, fire_p=False)
        step(seq - 1, 1, fire_g=False, fire_p=False)
        wait_out(0)
        wait_out(1)

    return gather_kernel


def _make_table(v, d):
    nb = v // 256
    per_w = nb // _NW
    extra = nb - per_w * _NW

    mesh = plsc.VectorSubcoreMesh(
        core_axis_name="c", subcore_axis_name="s",
        num_cores=_NC, num_subcores=_NS)

    @functools.partial(
        pl.kernel,
        mesh=mesh,
        compiler_params=pltpu.CompilerParams(
            use_tc_tiling_on_sc=True, needs_layout_passes=False),
        out_type=jax.ShapeDtypeStruct((v * d,), jnp.float32),
        scratch_types=[
            pltpu.VMEM((2, 64, 256), jnp.float32),
            pltpu.VMEM((64 * 256,), jnp.float32),
            pltpu.VMEM((64 * 256,), jnp.float32),
            pltpu.VMEM((4096,), jnp.float32),
            pltpu.SemaphoreType.DMA,
            pltpu.SemaphoreType.DMA,
            pltpu.SemaphoreType.DMA,
            pltpu.SemaphoreType.DMA,
            pltpu.SemaphoreType.DMA,
        ],
    )
    def tr_kernel(wt_hbm, tail_hbm, out_hbm, vin, vout0, vout1, tl_v,
                  isem0, isem1, osem0, osem1, tsem):
        wid = lax.axis_index("s") * _NC + lax.axis_index("c")
        base = wid * per_w + jnp.minimum(wid, extra)
        isem = (isem0, isem1)
        osem = (osem0, osem1)
        vout = (vout0, vout1)

        def fire_in(blk, a):
            pltpu.async_copy(
                wt_hbm.at[:, pl.ds((base + blk) * 256, 256)], vin.at[a],
                isem[a])

        def wait_in(a):
            pltpu.make_async_copy(
                wt_hbm.at[:, pl.ds(0, 256)], vin.at[a], isem[a]).wait()

        def fire_out(blk, a):
            pltpu.async_copy(
                vout[a],
                out_hbm.at[pl.ds((base + blk) * (64 * 256), 64 * 256)],
                osem[a])

        def wait_out(a):
            pltpu.make_async_copy(
                vout[a], out_hbm.at[pl.ds(0, 64 * 256)], osem[a]).wait()

        lanes = lax.iota(jnp.int32, _L)
        jidx = [lanes + jg * _L for jg in range(16)]
        j64 = [lax.shift_left(j, 6) for j in jidx]

        def transpose(a):
            def h0body(i, col):
                col2 = lax.bitwise_and(col + 1, d - 1)
                for cl, half in ((col, 0), (col, 1), (col2, 0), (col2, 1)):
                    vals = [plsc.load_gather(
                        vin.at[a], [cl, jidx[half * 8 + jg]])
                        for jg in range(8)]
                    for jg in range(8):
                        plsc.store_scatter(
                            vout[a], [j64[half * 8 + jg] + cl], vals[jg])
                return lax.bitwise_and(col2 + 1, d - 1)

            lax.fori_loop(0, d // 2, h0body, lanes)

        def step(blk, a, fire_nxt=True, wait_o=True):
            b = 1 - a
            if fire_nxt:
                pl.when(blk + 1 < per_w + (wid < extra))(
                    lambda: fire_in(blk + 1, b))
            wait_in(a)
            if wait_o:
                wait_out(a)
            transpose(a)
            fire_out(blk, a)

        @pl.when(wid == 0)
        def _():
            pltpu.async_copy(tail_hbm, tl_v, tsem)
            pltpu.make_async_copy(tail_hbm, tl_v, tsem).wait()
            pltpu.async_copy(
                tl_v, out_hbm.at[pl.ds((v // 256) * 256 * d, (v % 256) * d)],
                tsem)
            pltpu.make_async_copy(
                tl_v, out_hbm.at[pl.ds(0, (v % 256) * d)], tsem).wait()

        fire_in(0, 0)
        step(0, 0, wait_o=False)
        step(1, 1, wait_o=False)

        def body(i, carry):
            step(2 * i + 2, 0)
            step(2 * i + 3, 1)
            return carry

        lax.fori_loop(0, (per_w - 4) // 2, body, 0)

        step(per_w - 2, 0)
        step(per_w - 1, 1)

        @pl.when(wid < extra)
        def _():
            wait_in(0)
            wait_out(0)
            transpose(0)
            fire_out(per_w, 0)
            wait_out(0)
        pl.when(wid >= extra)(lambda: wait_out(0))
        wait_out(1)

    return tr_kernel


def kernel(input_ids, word_embeddings):
    batch, seq = input_ids.shape
    v, d = word_embeddings.shape
    idx_flat = input_ids.T.reshape(-1).astype(jnp.int32)
    wt = word_embeddings.T
    tail2 = word_embeddings[(v // 128) * 128:].reshape(-1)
    table2 = _make_table(v, d)(wt, tail2).reshape(v, d)
    out5 = _make(batch, seq, d)(idx_flat, table2)
    out = out5.transpose(2, 4, 0, 1, 3).reshape(batch, seq, d)
    return out
